# Initial kernel scaffold; baseline (speedup 1.0000x reference)
import jax
import jax.numpy as jnp
from jax.experimental import pallas as pl


def kernel(input, codebook):
    raise NotImplementedError("write your pallas kernel here")


# speedup vs baseline: 1.2092x; 1.2092x over previous
import jax
import jax.numpy as jnp
from jax import lax
from jax.experimental import pallas as pl

_K = 1024
_D = 64
_B = 16
_HW = 1024


def _vq_body(x_ref, cb_ref, idx_ref, emb_ref):
    xT = x_ref[0]
    cb = cb_ref[...]
    scoresT = lax.dot_general(cb, xT, (((1,), (0,)), ((), ())),
                              preferred_element_type=jnp.float32)
    x2 = jnp.sum(xT * xT, axis=0, keepdims=True)
    c2 = jnp.sum(cb * cb, axis=1, keepdims=True)
    d2 = (x2 + c2) - 2.0 * scoresT
    dist = jnp.sqrt(jnp.maximum(d2, 0.0))
    idx = jnp.argmin(dist, axis=0)
    idx_ref[0] = idx.reshape(1, _HW)
    onehotT = (lax.broadcasted_iota(jnp.int32, (_K, _HW), 0) == idx[None, :]
               ).astype(jnp.float32)
    embT = lax.dot_general(cb, onehotT, (((0,), (0,)), ((), ())),
                           preferred_element_type=jnp.float32,
                           precision=lax.Precision.HIGHEST)
    emb_ref[0] = embT


def kernel(input, codebook):
    inp = input.reshape(_B, _D, _HW)
    idx3, emb3 = pl.pallas_call(
        _vq_body,
        grid=(_B,),
        in_specs=[
            pl.BlockSpec((1, _D, _HW), lambda b: (b, 0, 0)),
            pl.BlockSpec((_K, _D), lambda b: (0, 0)),
        ],
        out_specs=[
            pl.BlockSpec((1, 1, _HW), lambda b: (b, 0, 0)),
            pl.BlockSpec((1, _D, _HW), lambda b: (b, 0, 0)),
        ],
        out_shape=[
            jax.ShapeDtypeStruct((_B, 1, _HW), jnp.int32),
            jax.ShapeDtypeStruct((_B, _D, _HW), jnp.float32),
        ],
    )(inp, codebook)
    embed = emb3.reshape(_B, _D, 32, 32)
    idxes = idx3.reshape(_B, 32, 32)
    return (embed, idxes)


# speedup vs baseline: 1.3477x; 1.1146x over previous
import jax
import jax.numpy as jnp
from jax import lax
from jax.experimental import pallas as pl

_K = 1024
_D = 64
_B = 16
_HW = 1024


def _vq_body(x_ref, cb_ref, idx_ref, emb_ref):
    xT = x_ref[0]
    cb = cb_ref[...]
    scoresT = lax.dot_general(cb, xT, (((1,), (0,)), ((), ())),
                              preferred_element_type=jnp.float32)
    x2 = jnp.sum(xT * xT, axis=0, keepdims=True)
    c2 = jnp.sum(cb * cb, axis=1, keepdims=True)
    d2 = (x2 + c2) - 2.0 * scoresT
    m2 = jnp.min(d2, axis=0, keepdims=True)
    m2c = jnp.maximum(m2, 0.0)
    s = jnp.sqrt(m2c)
    hb = lax.bitcast_convert_type(s * s, jnp.int32)
    hi = m2c
    for k in range(-4, 5):
        c = lax.bitcast_convert_type(hb + k, jnp.float32)
        ok = (c >= 0.0) & (jnp.sqrt(c) == s)
        hi = jnp.where(ok, jnp.maximum(hi, c), hi)
    hi = jnp.where(s > 0.0, hi, 0.0)
    kiota = lax.broadcasted_iota(jnp.int32, (_K, _HW), 0)
    idx = jnp.min(jnp.where(d2 <= hi, kiota, _K), axis=0)
    idx_ref[0] = idx.reshape(1, _HW)
    onehotT = (lax.broadcasted_iota(jnp.int32, (_K, _HW), 0) == idx[None, :]
               ).astype(jnp.float32)
    embT = lax.dot_general(cb, onehotT, (((0,), (0,)), ((), ())),
                           preferred_element_type=jnp.float32,
                           precision=lax.Precision.HIGHEST)
    emb_ref[0] = embT


def kernel(input, codebook):
    inp = input.reshape(_B, _D, _HW)
    idx3, emb3 = pl.pallas_call(
        _vq_body,
        grid=(_B,),
        in_specs=[
            pl.BlockSpec((1, _D, _HW), lambda b: (b, 0, 0)),
            pl.BlockSpec((_K, _D), lambda b: (0, 0)),
        ],
        out_specs=[
            pl.BlockSpec((1, 1, _HW), lambda b: (b, 0, 0)),
            pl.BlockSpec((1, _D, _HW), lambda b: (b, 0, 0)),
        ],
        out_shape=[
            jax.ShapeDtypeStruct((_B, 1, _HW), jnp.int32),
            jax.ShapeDtypeStruct((_B, _D, _HW), jnp.float32),
        ],
    )(inp, codebook)
    embed = emb3.reshape(_B, _D, 32, 32)
    idxes = idx3.reshape(_B, 32, 32)
    return (embed, idxes)
